# Initial kernel scaffold; baseline (speedup 1.0000x reference)
import jax
import jax.numpy as jnp
from jax.experimental import pallas as pl


def kernel(query, key, value, Wq_u, Wq_v, Uq_u, Uq_v, Wk_u, Wk_v, Uk_u, Uk_v, Wv_u, Wv_v, lsh_vecs, Wo):
    raise NotImplementedError("write your pallas kernel here")


# speedup vs baseline: 189.7814x; 189.7814x over previous
import jax
import jax.numpy as jnp
from jax.experimental import pallas as pl

_B, _S, _D = 1, 2048, 768
_H, _DK, _R = 12, 64, 16
_BW, _NB = 4.0, 64
_P = 8

_BQ = 256
_BS = 512


def _prep_body(wq_u, wq_v, uq_u, uq_v, wk_u, wk_v, uk_u, uk_v, wv_u, wv_v,
               wq_eff, wk_eff, wv_eff):
    f32 = jnp.float32
    a = jnp.dot(wq_v[0], uq_u[0], preferred_element_type=f32)
    b = jnp.dot(a, uq_v[0], preferred_element_type=f32)
    wq_eff[0] = jnp.dot(wq_u[0], b, preferred_element_type=f32)
    a = jnp.dot(wk_v[0], uk_u[0], preferred_element_type=f32)
    b = jnp.dot(a, uk_v[0], preferred_element_type=f32)
    wk_eff[0] = jnp.dot(wk_u[0], b, preferred_element_type=f32)
    wv_eff[0] = jnp.dot(wv_u[0], wv_v[0], preferred_element_type=f32)


def _proj_body(q_ref, k_ref, v_ref, wq_ref, wk_ref, wv_ref,
               qu_ref, ku_ref, vu_ref):
    f32 = jnp.float32
    qu_ref[0] = jnp.dot(q_ref[0], wq_ref[0], preferred_element_type=f32)
    ku_ref[0] = jnp.dot(k_ref[0], wk_ref[0], preferred_element_type=f32)
    vu_ref[0] = jnp.dot(v_ref[0], wv_ref[0], preferred_element_type=f32)


def _attn_body(qb_ref, qf_ref, kf_ref, vf_ref, lsh_ref, out_ref):
    f32 = jnp.float32
    h = pl.program_id(0)
    qb = qb_ref[0]
    kf = kf_ref[0]
    vf = vf_ref[0]
    lsh = lsh_ref[h, :]

    qdot = jnp.dot(qb, lsh[:, None], preferred_element_type=f32)
    kdot = jnp.dot(kf, lsh[:, None], preferred_element_type=f32)
    qh = jnp.mod(jnp.floor(qdot * (1.0 / _BW)), float(_NB))
    kh = jnp.mod(jnp.floor(kdot * (1.0 / _BW)), float(_NB))

    qs = jnp.where(qb > 0, 1.0, -1.0).astype(f32)
    ks = jnp.where(kf > 0, 1.0, -1.0).astype(f32)
    sgn = jnp.dot(qs, ks.T, preferred_element_type=f32)
    sim = jnp.dot(qb, kf.T, preferred_element_type=f32)

    qf8 = jnp.where(qf_ref[0][:, :_P] > 0, 1.0, -1.0).astype(f32)
    ks8 = ks[:, :_P]
    ins = jnp.sum(qf8 * ks8, axis=1) > (_P - 0.5)

    mask = (sgn > (_DK - 0.5)) & (qh == kh.T) & ins[None, :] & (sim > -1e8)

    s = jnp.maximum(sim * 0.125, -1e4)
    m = jnp.max(jnp.where(mask, s, -1e30), axis=1, keepdims=True)
    e = jnp.where(mask, jnp.exp(s - m), 0.0)
    den = jnp.maximum(jnp.sum(e, axis=1, keepdims=True), 1e-9)
    p = e / den
    out_ref[0] = jnp.dot(p, vf, preferred_element_type=f32)


def _oproj_body(ho_ref, wo_ref, out_ref):
    h = pl.program_id(1)
    part = jnp.dot(ho_ref[0], wo_ref[...], preferred_element_type=jnp.float32)

    @pl.when(h == 0)
    def _init():
        out_ref[...] = part

    @pl.when(h != 0)
    def _acc():
        out_ref[...] += part


def kernel(query, key, value, Wq_u, Wq_v, Uq_u, Uq_v, Wk_u, Wk_v, Uk_u, Uk_v,
           Wv_u, Wv_v, lsh_vecs, Wo):
    f32 = jnp.float32

    wq_eff, wk_eff, wv_eff = pl.pallas_call(
        _prep_body,
        grid=(_H,),
        in_specs=[
            pl.BlockSpec((1, _D, _R), lambda h: (h, 0, 0)),
            pl.BlockSpec((1, _R, _DK), lambda h: (h, 0, 0)),
            pl.BlockSpec((1, _DK, _R), lambda h: (h, 0, 0)),
            pl.BlockSpec((1, _R, _DK), lambda h: (h, 0, 0)),
            pl.BlockSpec((1, _D, _R), lambda h: (h, 0, 0)),
            pl.BlockSpec((1, _R, _DK), lambda h: (h, 0, 0)),
            pl.BlockSpec((1, _DK, _R), lambda h: (h, 0, 0)),
            pl.BlockSpec((1, _R, _DK), lambda h: (h, 0, 0)),
            pl.BlockSpec((1, _D, _R), lambda h: (h, 0, 0)),
            pl.BlockSpec((1, _R, _DK), lambda h: (h, 0, 0)),
        ],
        out_specs=[pl.BlockSpec((1, _D, _DK), lambda h: (h, 0, 0))] * 3,
        out_shape=[jax.ShapeDtypeStruct((_H, _D, _DK), f32)] * 3,
    )(Wq_u, Wq_v, Uq_u, Uq_v, Wk_u, Wk_v, Uk_u, Uk_v, Wv_u, Wv_v)

    q_up, k_up, v_up = pl.pallas_call(
        _proj_body,
        grid=(_H,),
        in_specs=[
            pl.BlockSpec((1, _S, _D), lambda h: (0, 0, 0)),
            pl.BlockSpec((1, _S, _D), lambda h: (0, 0, 0)),
            pl.BlockSpec((1, _S, _D), lambda h: (0, 0, 0)),
            pl.BlockSpec((1, _D, _DK), lambda h: (h, 0, 0)),
            pl.BlockSpec((1, _D, _DK), lambda h: (h, 0, 0)),
            pl.BlockSpec((1, _D, _DK), lambda h: (h, 0, 0)),
        ],
        out_specs=[pl.BlockSpec((1, _S, _DK), lambda h: (h, 0, 0))] * 3,
        out_shape=[jax.ShapeDtypeStruct((_H, _S, _DK), f32)] * 3,
    )(query, key, value, wq_eff, wk_eff, wv_eff)

    n_qb = _S // _BQ
    head_out = pl.pallas_call(
        _attn_body,
        grid=(_H, n_qb),
        in_specs=[
            pl.BlockSpec((1, _BQ, _DK), lambda h, i: (h, i, 0)),
            pl.BlockSpec((1, _S, _DK), lambda h, i: (h, 0, 0)),
            pl.BlockSpec((1, _S, _DK), lambda h, i: (h, 0, 0)),
            pl.BlockSpec((1, _S, _DK), lambda h, i: (h, 0, 0)),
            pl.BlockSpec((_H, _DK), lambda h, i: (0, 0)),
        ],
        out_specs=pl.BlockSpec((1, _BQ, _DK), lambda h, i: (h, i, 0)),
        out_shape=jax.ShapeDtypeStruct((_H, _S, _DK), f32),
    )(q_up, q_up, k_up, v_up, lsh_vecs)

    n_rb = _S // _BS
    out = pl.pallas_call(
        _oproj_body,
        grid=(n_rb, _H),
        in_specs=[
            pl.BlockSpec((1, _BS, _DK), lambda i, h: (h, i, 0)),
            pl.BlockSpec((_DK, _D), lambda i, h: (h, 0)),
        ],
        out_specs=pl.BlockSpec((_BS, _D), lambda i, h: (i, 0)),
        out_shape=jax.ShapeDtypeStruct((_S, _D), f32),
    )(head_out, Wo)

    return out.reshape(_B, _S, _D)


# speedup vs baseline: 427.5739x; 2.2530x over previous
import jax
import jax.numpy as jnp
from jax import lax
from jax.experimental import pallas as pl

_B, _S, _D = 1, 2048, 768
_H, _DK, _R = 12, 64, 16
_BW, _NB = 4.0, 64
_P = 8
_NBITS = 6
_KSIG = 128
_FULL = _DK + _NBITS + 1

_BQ = 512
_BS = 512


def _signature(x_up, bucket, flag_col):
    f32 = jnp.float32
    rows = x_up.shape[0]
    signs = jnp.where(x_up > 0, 1.0, -1.0).astype(f32)
    pows = jnp.exp2(
        lax.broadcasted_iota(jnp.int32, (1, _NBITS), 1).astype(f32))
    bits = jnp.mod(jnp.floor(bucket / pows), 2.0)
    bsign = 2.0 * bits - 1.0
    pad = jnp.zeros((rows, _KSIG - _FULL), f32)
    sig = jnp.concatenate([signs, bsign, flag_col, pad], axis=1)
    return sig.astype(jnp.bfloat16)


def _proj_body(q_ref, k_ref, v_ref,
               wq_u, wq_v, uq_u, uq_v, wk_u, wk_v, uk_u, uk_v, wv_u, wv_v,
               lsh_ref,
               qu_ref, ku_ref, vu_ref, qsig_ref, ksig_ref):
    f32 = jnp.float32
    h = pl.program_id(0)

    a = jnp.dot(wq_v[0], uq_u[0], preferred_element_type=f32)
    b = jnp.dot(a, uq_v[0], preferred_element_type=f32)
    wq_eff = jnp.dot(wq_u[0], b, preferred_element_type=f32)
    a = jnp.dot(wk_v[0], uk_u[0], preferred_element_type=f32)
    b = jnp.dot(a, uk_v[0], preferred_element_type=f32)
    wk_eff = jnp.dot(wk_u[0], b, preferred_element_type=f32)
    wv_eff = jnp.dot(wv_u[0], wv_v[0], preferred_element_type=f32)

    q_up = jnp.dot(q_ref[0], wq_eff, preferred_element_type=f32)
    k_up = jnp.dot(k_ref[0], wk_eff, preferred_element_type=f32)
    v_up = jnp.dot(v_ref[0], wv_eff, preferred_element_type=f32)
    qu_ref[0] = q_up
    ku_ref[0] = k_up
    vu_ref[0] = v_up

    lsh = lsh_ref[h, :]
    qb = jnp.mod(jnp.floor(
        jnp.dot(q_up, lsh[:, None], preferred_element_type=f32) * (1.0 / _BW)),
        float(_NB))
    kb = jnp.mod(jnp.floor(
        jnp.dot(k_up, lsh[:, None], preferred_element_type=f32) * (1.0 / _BW)),
        float(_NB))

    agree = jnp.where(q_up[:, :_P] > 0, 1.0, -1.0) * \
        jnp.where(k_up[:, :_P] > 0, 1.0, -1.0)
    ins = (jnp.sum(agree, axis=1, keepdims=True) > (_P - 0.5)).astype(f32)

    ones = jnp.ones((_S, 1), f32)
    qsig_ref[0] = _signature(q_up, qb, ones)
    ksig_ref[0] = _signature(k_up, kb, ins)


def _attn_body(qsig_ref, ksig_ref, qb_ref, kf_ref, vf_ref, out_ref):
    f32 = jnp.float32
    score = jnp.dot(qsig_ref[0], ksig_ref[0].T,
                    preferred_element_type=f32)
    got = jnp.max(score) > (_FULL - 0.5)

    @pl.when(got)
    def _slow():
        qb = qb_ref[0]
        kf = kf_ref[0]
        sim = jnp.dot(qb, kf.T, preferred_element_type=f32)
        mask = (score > (_FULL - 0.5)) & (sim > -1e8)
        s = jnp.maximum(sim * 0.125, -1e4)
        m = jnp.max(jnp.where(mask, s, -1e30), axis=1, keepdims=True)
        e = jnp.where(mask, jnp.exp(s - m), 0.0)
        den = jnp.maximum(jnp.sum(e, axis=1, keepdims=True), 1e-9)
        p = e / den
        out_ref[0] = jnp.dot(p, vf_ref[0], preferred_element_type=f32)

    @pl.when(jnp.logical_not(got))
    def _fast():
        out_ref[0] = jnp.zeros((_BQ, _DK), f32)


def _oproj_body(ho_ref, wo_ref, out_ref):
    h = pl.program_id(1)
    ho = ho_ref[0]

    @pl.when(h == 0)
    def _init():
        out_ref[...] = jnp.zeros(out_ref.shape, jnp.float32)

    nz = jnp.any(ho != 0.0)

    @pl.when(nz)
    def _acc():
        out_ref[...] += jnp.dot(ho, wo_ref[...],
                                preferred_element_type=jnp.float32)


def kernel(query, key, value, Wq_u, Wq_v, Uq_u, Uq_v, Wk_u, Wk_v, Uk_u, Uk_v,
           Wv_u, Wv_v, lsh_vecs, Wo):
    f32 = jnp.float32

    wspec = [
        pl.BlockSpec((1, _D, _R), lambda h: (h, 0, 0)),
        pl.BlockSpec((1, _R, _DK), lambda h: (h, 0, 0)),
        pl.BlockSpec((1, _DK, _R), lambda h: (h, 0, 0)),
        pl.BlockSpec((1, _R, _DK), lambda h: (h, 0, 0)),
        pl.BlockSpec((1, _D, _R), lambda h: (h, 0, 0)),
        pl.BlockSpec((1, _R, _DK), lambda h: (h, 0, 0)),
        pl.BlockSpec((1, _DK, _R), lambda h: (h, 0, 0)),
        pl.BlockSpec((1, _R, _DK), lambda h: (h, 0, 0)),
        pl.BlockSpec((1, _D, _R), lambda h: (h, 0, 0)),
        pl.BlockSpec((1, _R, _DK), lambda h: (h, 0, 0)),
    ]
    q_up, k_up, v_up, qsig, ksig = pl.pallas_call(
        _proj_body,
        grid=(_H,),
        in_specs=[
            pl.BlockSpec((1, _S, _D), lambda h: (0, 0, 0)),
            pl.BlockSpec((1, _S, _D), lambda h: (0, 0, 0)),
            pl.BlockSpec((1, _S, _D), lambda h: (0, 0, 0)),
        ] + wspec + [pl.BlockSpec((_H, _DK), lambda h: (0, 0))],
        out_specs=[pl.BlockSpec((1, _S, _DK), lambda h: (h, 0, 0))] * 3 +
                  [pl.BlockSpec((1, _S, _KSIG), lambda h: (h, 0, 0))] * 2,
        out_shape=[jax.ShapeDtypeStruct((_H, _S, _DK), f32)] * 3 +
                  [jax.ShapeDtypeStruct((_H, _S, _KSIG), jnp.bfloat16)] * 2,
    )(query, key, value, Wq_u, Wq_v, Uq_u, Uq_v, Wk_u, Wk_v, Uk_u, Uk_v,
      Wv_u, Wv_v, lsh_vecs)

    n_qb = _S // _BQ
    head_out = pl.pallas_call(
        _attn_body,
        grid=(_H, n_qb),
        in_specs=[
            pl.BlockSpec((1, _BQ, _KSIG), lambda h, i: (h, i, 0)),
            pl.BlockSpec((1, _S, _KSIG), lambda h, i: (h, 0, 0)),
            pl.BlockSpec((1, _BQ, _DK), lambda h, i: (h, i, 0)),
            pl.BlockSpec((1, _S, _DK), lambda h, i: (h, 0, 0)),
            pl.BlockSpec((1, _S, _DK), lambda h, i: (h, 0, 0)),
        ],
        out_specs=pl.BlockSpec((1, _BQ, _DK), lambda h, i: (h, i, 0)),
        out_shape=jax.ShapeDtypeStruct((_H, _S, _DK), f32),
    )(qsig, ksig, q_up, k_up, v_up)

    n_rb = _S // _BS
    out = pl.pallas_call(
        _oproj_body,
        grid=(n_rb, _H),
        in_specs=[
            pl.BlockSpec((1, _BS, _DK), lambda i, h: (h, i, 0)),
            pl.BlockSpec((_DK, _D), lambda i, h: (h, 0)),
        ],
        out_specs=pl.BlockSpec((_BS, _D), lambda i, h: (i, 0)),
        out_shape=jax.ShapeDtypeStruct((_S, _D), f32),
    )(head_out, Wo)

    return out.reshape(_B, _S, _D)


# speedup vs baseline: 509.9645x; 1.1927x over previous
import jax
import jax.numpy as jnp
from jax import lax
from jax.experimental import pallas as pl

_B, _S, _D = 1, 2048, 768
_H, _DK, _R = 12, 64, 16
_BW, _NB = 4.0, 64
_P = 8
_NBITS = 6
_KSIG = 128
_FULL = _DK + _NBITS + 1

_BQ = 512
_BS = 1024


def _signature(x_up, bucket, flag_col):
    f32 = jnp.float32
    rows = x_up.shape[0]
    signs = jnp.where(x_up > 0, 1.0, -1.0).astype(f32)
    shifts = lax.broadcasted_iota(jnp.int32, (1, _NBITS), 1)
    ibits = jnp.bitwise_and(
        jnp.right_shift(bucket.astype(jnp.int32), shifts), 1)
    bsign = (2 * ibits - 1).astype(f32)
    pad = jnp.zeros((rows, _KSIG - _FULL), f32)
    sig = jnp.concatenate([signs, bsign, flag_col, pad], axis=1)
    return sig.astype(jnp.int8)


def _proj_body(q_ref, k_ref, v_ref,
               wq_u, wq_v, uq_u, uq_v, wk_u, wk_v, uk_u, uk_v, wv_u, wv_v,
               lsh_ref,
               qu_ref, ku_ref, vu_ref, qsig_ref, ksig_ref):
    f32 = jnp.float32
    h = pl.program_id(0)

    a = jnp.dot(wq_v[0], uq_u[0], preferred_element_type=f32)
    b = jnp.dot(a, uq_v[0], preferred_element_type=f32)
    wq_eff = jnp.dot(wq_u[0], b, preferred_element_type=f32)
    a = jnp.dot(wk_v[0], uk_u[0], preferred_element_type=f32)
    b = jnp.dot(a, uk_v[0], preferred_element_type=f32)
    wk_eff = jnp.dot(wk_u[0], b, preferred_element_type=f32)
    wv_eff = jnp.dot(wv_u[0], wv_v[0], preferred_element_type=f32)

    q_up = jnp.dot(q_ref[0], wq_eff, preferred_element_type=f32)
    k_up = jnp.dot(k_ref[0], wk_eff, preferred_element_type=f32)
    v_up = jnp.dot(v_ref[0], wv_eff, preferred_element_type=f32)
    qu_ref[0] = q_up
    ku_ref[0] = k_up
    vu_ref[0] = v_up

    lsh = lsh_ref[h, :]
    qb = jnp.mod(jnp.floor(
        jnp.dot(q_up, lsh[:, None], preferred_element_type=f32) * (1.0 / _BW)),
        float(_NB))
    kb = jnp.mod(jnp.floor(
        jnp.dot(k_up, lsh[:, None], preferred_element_type=f32) * (1.0 / _BW)),
        float(_NB))

    agree = jnp.where(q_up[:, :_P] > 0, 1.0, -1.0) * \
        jnp.where(k_up[:, :_P] > 0, 1.0, -1.0)
    ins = (jnp.sum(agree, axis=1, keepdims=True) > (_P - 0.5)).astype(f32)

    ones = jnp.ones((_S, 1), f32)
    qsig_ref[0] = _signature(q_up, qb, ones)
    ksig_ref[0] = _signature(k_up, kb, ins)


def _attn_body(qsig_ref, ksig_ref, qb_ref, kf_ref, vf_ref, out_ref):
    f32 = jnp.float32
    score = jnp.dot(qsig_ref[0], ksig_ref[0].T,
                    preferred_element_type=jnp.int32)
    got = jnp.max(score) > (_FULL - 1)

    @pl.when(got)
    def _slow():
        qb = qb_ref[0]
        kf = kf_ref[0]
        sim = jnp.dot(qb, kf.T, preferred_element_type=f32)
        mask = (score > (_FULL - 1)) & (sim > -1e8)
        s = jnp.maximum(sim * 0.125, -1e4)
        m = jnp.max(jnp.where(mask, s, -1e30), axis=1, keepdims=True)
        e = jnp.where(mask, jnp.exp(s - m), 0.0)
        den = jnp.maximum(jnp.sum(e, axis=1, keepdims=True), 1e-9)
        p = e / den
        out_ref[0] = jnp.dot(p, vf_ref[0], preferred_element_type=f32)

    @pl.when(jnp.logical_not(got))
    def _fast():
        out_ref[0] = jnp.zeros((_BQ, _DK), f32)


def _oproj_body(ho_ref, wo_ref, out_ref):
    h = pl.program_id(1)
    ho = ho_ref[0]

    @pl.when(h == 0)
    def _init():
        out_ref[...] = jnp.zeros(out_ref.shape, jnp.float32)

    nz = jnp.any(ho != 0.0)

    @pl.when(nz)
    def _acc():
        out_ref[...] += jnp.dot(ho, wo_ref[...],
                                preferred_element_type=jnp.float32)


def kernel(query, key, value, Wq_u, Wq_v, Uq_u, Uq_v, Wk_u, Wk_v, Uk_u, Uk_v,
           Wv_u, Wv_v, lsh_vecs, Wo):
    f32 = jnp.float32

    wspec = [
        pl.BlockSpec((1, _D, _R), lambda h: (h, 0, 0)),
        pl.BlockSpec((1, _R, _DK), lambda h: (h, 0, 0)),
        pl.BlockSpec((1, _DK, _R), lambda h: (h, 0, 0)),
        pl.BlockSpec((1, _R, _DK), lambda h: (h, 0, 0)),
        pl.BlockSpec((1, _D, _R), lambda h: (h, 0, 0)),
        pl.BlockSpec((1, _R, _DK), lambda h: (h, 0, 0)),
        pl.BlockSpec((1, _DK, _R), lambda h: (h, 0, 0)),
        pl.BlockSpec((1, _R, _DK), lambda h: (h, 0, 0)),
        pl.BlockSpec((1, _D, _R), lambda h: (h, 0, 0)),
        pl.BlockSpec((1, _R, _DK), lambda h: (h, 0, 0)),
    ]
    q_up, k_up, v_up, qsig, ksig = pl.pallas_call(
        _proj_body,
        grid=(_H,),
        in_specs=[
            pl.BlockSpec((1, _S, _D), lambda h: (0, 0, 0)),
            pl.BlockSpec((1, _S, _D), lambda h: (0, 0, 0)),
            pl.BlockSpec((1, _S, _D), lambda h: (0, 0, 0)),
        ] + wspec + [pl.BlockSpec((_H, _DK), lambda h: (0, 0))],
        out_specs=[pl.BlockSpec((1, _S, _DK), lambda h: (h, 0, 0))] * 3 +
                  [pl.BlockSpec((1, _S, _KSIG), lambda h: (h, 0, 0))] * 2,
        out_shape=[jax.ShapeDtypeStruct((_H, _S, _DK), f32)] * 3 +
                  [jax.ShapeDtypeStruct((_H, _S, _KSIG), jnp.int8)] * 2,
    )(query, key, value, Wq_u, Wq_v, Uq_u, Uq_v, Wk_u, Wk_v, Uk_u, Uk_v,
      Wv_u, Wv_v, lsh_vecs)

    n_qb = _S // _BQ
    head_out = pl.pallas_call(
        _attn_body,
        grid=(_H, n_qb),
        in_specs=[
            pl.BlockSpec((1, _BQ, _KSIG), lambda h, i: (h, i, 0)),
            pl.BlockSpec((1, _S, _KSIG), lambda h, i: (h, 0, 0)),
            pl.BlockSpec((1, _BQ, _DK), lambda h, i: (h, i, 0)),
            pl.BlockSpec((1, _S, _DK), lambda h, i: (h, 0, 0)),
            pl.BlockSpec((1, _S, _DK), lambda h, i: (h, 0, 0)),
        ],
        out_specs=pl.BlockSpec((1, _BQ, _DK), lambda h, i: (h, i, 0)),
        out_shape=jax.ShapeDtypeStruct((_H, _S, _DK), f32),
    )(qsig, ksig, q_up, k_up, v_up)

    n_rb = _S // _BS
    out = pl.pallas_call(
        _oproj_body,
        grid=(n_rb, _H),
        in_specs=[
            pl.BlockSpec((1, _BS, _DK), lambda i, h: (h, i, 0)),
            pl.BlockSpec((_DK, _D), lambda i, h: (h, 0)),
        ],
        out_specs=pl.BlockSpec((_BS, _D), lambda i, h: (i, 0)),
        out_shape=jax.ShapeDtypeStruct((_S, _D), f32),
    )(head_out, Wo)

    return out.reshape(_B, _S, _D)
